# baseline (speedup 1.0000x reference)
import jax
import jax.numpy as jnp
from jax.experimental import pallas as pl
from jax.experimental.pallas import tpu as pltpu

_N = 4096
_IN = 512
_H = 512
_L = 2
_BLK = 256


def _stage1_body(f_ref, wg_ref, ww_ref, wy_ref, seq_ref, v_ref):
    f = f_ref[...]
    wg = wg_ref[0]
    seq_ref[0] = jax.lax.dot_general(
        f, wg, (((1,), (1,)), ((), ())),
        preferred_element_type=jnp.float32).astype(jnp.bfloat16)
    ww = ww_ref[0]
    wy = wy_ref[0, 0]
    v_ref[0, 0] = jnp.sum(ww * wy[:, None], axis=0)


def _stage2_body(adj_ref, seq_ref, bg_ref, v_ref, by_ref, final_ref, layers_ref):
    embs = []
    for l in range(_L):
        a = adj_ref[l].astype(jnp.bfloat16)
        s = seq_ref[l]
        e = jax.lax.dot_general(
            a, s, (((1,), (0,)), ((), ())), preferred_element_type=jnp.float32)
        e = jnp.maximum(e + bg_ref[l, 0], 0.0)
        layers_ref[l] = e
        embs.append(e)
    ws = []
    for l in range(_L):
        v = v_ref[l, 0]
        s = jnp.sum(embs[l] * v, axis=1, keepdims=True) + by_ref[0, l]
        ws.append(jnp.exp(jnp.tanh(s)))
    inv = 1.0 / (ws[0] + ws[1])
    final_ref[...] = (ws[0] * embs[0] + ws[1] * embs[1]) * inv


def kernel(features, adj_list, W_gcn, b_gcn, W_w, W_y, b_y, sparse):
    f = features[0]
    adj = adj_list[:, 0]
    wy3 = W_y.reshape(_L, 1, _H)
    bg3 = b_gcn.reshape(_L, 1, _H)
    by2 = b_y.reshape(1, _L)

    seq, v = pl.pallas_call(
        _stage1_body,
        grid=(_L,),
        in_specs=[
            pl.BlockSpec((_N, _IN), lambda l: (0, 0)),
            pl.BlockSpec((1, _H, _IN), lambda l: (l, 0, 0)),
            pl.BlockSpec((1, _H, _H), lambda l: (l, 0, 0)),
            pl.BlockSpec((1, 1, _H), lambda l: (l, 0, 0)),
        ],
        out_specs=[
            pl.BlockSpec((1, _N, _H), lambda l: (l, 0, 0)),
            pl.BlockSpec((1, 1, _H), lambda l: (l, 0, 0)),
        ],
        out_shape=[
            jax.ShapeDtypeStruct((_L, _N, _H), jnp.bfloat16),
            jax.ShapeDtypeStruct((_L, 1, _H), jnp.float32),
        ],
    )(f, W_gcn, W_w, wy3)

    nb = _N // _BLK
    final, layers = pl.pallas_call(
        _stage2_body,
        grid=(nb,),
        in_specs=[
            pl.BlockSpec((_L, _BLK, _N), lambda b: (0, b, 0)),
            pl.BlockSpec((_L, _N, _H), lambda b: (0, 0, 0)),
            pl.BlockSpec((_L, 1, _H), lambda b: (0, 0, 0)),
            pl.BlockSpec((_L, 1, _H), lambda b: (0, 0, 0)),
            pl.BlockSpec((1, _L), lambda b: (0, 0)),
        ],
        out_specs=[
            pl.BlockSpec((_BLK, _H), lambda b: (b, 0)),
            pl.BlockSpec((_L, _BLK, _H), lambda b: (0, b, 0)),
        ],
        out_shape=[
            jax.ShapeDtypeStruct((_N, _H), jnp.float32),
            jax.ShapeDtypeStruct((_L, _N, _H), jnp.float32),
        ],
        compiler_params=pltpu.CompilerParams(
            dimension_semantics=("parallel",)),
    )(adj, seq, bg3, v, by2)

    return (final, layers)


# speedup vs baseline: 1.0839x; 1.0839x over previous
import jax
import jax.numpy as jnp
from jax.experimental import pallas as pl
from jax.experimental.pallas import tpu as pltpu

_N = 4096
_IN = 512
_H = 512
_L = 2
_BLK = 256


def _body(f_ref, wg_ref, ww_ref, wy_ref, bg_ref, by_ref,
          adj_ref, final_ref, layers_ref, seq_s, v_s):
    @pl.when(pl.program_id(0) == 0)
    def _prologue():
        f = f_ref[...].astype(jnp.bfloat16)
        for l in range(_L):
            wg = wg_ref[l].astype(jnp.bfloat16)
            seq_s[l] = jax.lax.dot_general(
                f, wg, (((1,), (1,)), ((), ())),
                preferred_element_type=jnp.float32).astype(jnp.bfloat16)
            ww = ww_ref[l]
            wy = wy_ref[l, 0]
            v_s[l, 0:1, :] = jnp.sum(ww * wy[:, None], axis=0)[None]

    embs = []
    for l in range(_L):
        a = adj_ref[l].astype(jnp.bfloat16)
        e = jax.lax.dot_general(
            a, seq_s[l], (((1,), (0,)), ((), ())),
            preferred_element_type=jnp.float32)
        e = jnp.maximum(e + bg_ref[l, 0], 0.0)
        layers_ref[l] = e
        embs.append(e)
    ws = []
    for l in range(_L):
        v = v_s[l, 0]
        s = jnp.sum(embs[l] * v, axis=1, keepdims=True) + by_ref[0, l]
        ws.append(jnp.exp(jnp.tanh(s)))
    inv = 1.0 / (ws[0] + ws[1])
    final_ref[...] = (ws[0] * embs[0] + ws[1] * embs[1]) * inv


def kernel(features, adj_list, W_gcn, b_gcn, W_w, W_y, b_y, sparse):
    f = features[0]
    adj = adj_list[:, 0]
    wy3 = W_y.reshape(_L, 1, _H)
    bg3 = b_gcn.reshape(_L, 1, _H)
    by2 = b_y.reshape(1, _L)

    nb = _N // _BLK
    final, layers = pl.pallas_call(
        _body,
        grid=(nb,),
        in_specs=[
            pl.BlockSpec((_N, _IN), lambda b: (0, 0)),
            pl.BlockSpec((_L, _H, _IN), lambda b: (0, 0, 0)),
            pl.BlockSpec((_L, _H, _H), lambda b: (0, 0, 0)),
            pl.BlockSpec((_L, 1, _H), lambda b: (0, 0, 0)),
            pl.BlockSpec((_L, 1, _H), lambda b: (0, 0, 0)),
            pl.BlockSpec((1, _L), lambda b: (0, 0)),
            pl.BlockSpec((_L, _BLK, _N), lambda b: (0, b, 0)),
        ],
        out_specs=[
            pl.BlockSpec((_BLK, _H), lambda b: (b, 0)),
            pl.BlockSpec((_L, _BLK, _H), lambda b: (0, b, 0)),
        ],
        out_shape=[
            jax.ShapeDtypeStruct((_N, _H), jnp.float32),
            jax.ShapeDtypeStruct((_L, _N, _H), jnp.float32),
        ],
        scratch_shapes=[
            pltpu.VMEM((_L, _N, _H), jnp.bfloat16),
            pltpu.VMEM((_L, 8, _H), jnp.float32),
        ],
    )(f, W_gcn, W_w, wy3, bg3, by2, adj)

    return (final, layers)
